# baseline (speedup 1.0000x reference)
import jax
import jax.numpy as jnp
from jax.experimental import pallas as pl
from jax.experimental.pallas import tpu as pltpu


def _conv_body(xv_hbm, w_ref, b_ref, o_ref, xbufT, uxr, sems):
    _, Co, TD2, _ = o_ref.shape
    TD = TD2 // 2
    TDp, C, HW = xbufT.shape
    H = 16
    W = 16
    n = pl.program_id(0)
    t = pl.program_id(1)
    D = pl.num_programs(1) * TD

    for dd in range(TDp):
        dg = t * TD + dd - 1
        valid = jnp.logical_and(dg >= 0, dg < D)

        @pl.when(valid)
        def _(dd=dd, dg=dg):
            pltpu.make_async_copy(xv_hbm.at[n, :, dg], xbufT.at[dd],
                                  sems.at[dd]).start()

        @pl.when(jnp.logical_not(valid))
        def _(dd=dd):
            xbufT[dd] = jnp.zeros((C, HW), jnp.float32)

    @pl.when(jnp.logical_and(n == 0, t == 0))
    def _():
        uxr[...] = jnp.zeros(uxr.shape, uxr.dtype)

    for dd in range(TDp):
        dg = t * TD + dd - 1
        valid = jnp.logical_and(dg >= 0, dg < D)

        @pl.when(valid)
        def _(dd=dd):
            pltpu.make_async_copy(xv_hbm.at[n, :, 0], xbufT.at[dd],
                                  sems.at[dd]).wait()

        xx = xbufT[dd].astype(jnp.bfloat16)
        xx2 = jnp.stack([xx, xx], axis=2).reshape(C, 2 * HW)
        uxr[dd, :, 33:545] = xx2.reshape(C, H, 32).reshape(C, 512)

    lane = jax.lax.broadcasted_iota(jnp.int32, (C, 512), 1)
    m_lo = (lane % 32) != 0
    m_hi = (lane % 32) != 31

    def matmul(a, b, dl):
        pieces = []
        for rd in range(2):
            slab = a + rd + dl
            for rh in range(2):
                for kw in range(3):
                    st = 32 * (b + rh) + kw
                    p = uxr[slab, :, st:st + 512]
                    if kw == 0:
                        p = jnp.where(m_lo, p, jnp.bfloat16(0))
                    elif kw == 2:
                        p = jnp.where(m_hi, p, jnp.bfloat16(0))
                    pieces.append(p)
        patchT = jnp.concatenate(pieces, axis=0)
        y = jax.lax.dot_general(w_ref[2 * a + b], patchT,
                                (((1,), (0,)), ((), ())),
                                preferred_element_type=jnp.float32)
        return y + b_ref[...]

    zs = []
    for dl in range(TD):
        for a in range(2):
            y0 = matmul(a, 0, dl)
            y1 = matmul(a, 1, dl)
            z = jnp.stack([y0.reshape(Co, H, 32), y1.reshape(Co, H, 32)],
                          axis=2).reshape(Co, 1024)
            zs.append(z)
    o_ref[0] = jnp.stack(zs, axis=1)


def _invert_w_parity(we):
    C = we.shape[1] // 12
    Co = we.shape[2] // 2
    wr = we.reshape(2, 2, 2, 2, 3, C, 2, Co)
    w0 = wr[:, :, :, :, 0, :, 0, :]
    w2 = wr[:, :, :, :, 2, :, 1, :]
    w1 = wr[:, :, :, :, 1, :, 0, :] - w2
    wds = jnp.stack([w0, w1, w2], axis=4)
    return wds.transpose(0, 1, 6, 2, 3, 4, 5).reshape(4, Co, 12 * C)


def kernel(hidden_states, we, bias2):
    N, C, D, H, W = hidden_states.shape
    Co = we.shape[2] // 2
    TD = 4
    DT = D // TD
    xv = hidden_states.reshape(N, C, D, H * W)
    wlhs = _invert_w_parity(we).astype(jnp.bfloat16)
    bias_bc = jnp.broadcast_to(bias2[0, :Co].reshape(Co, 1), (Co, 512))

    flops = 2 * 4 * N * D * H * W * 12 * C * 2 * Co
    bytes_accessed = (xv.size * 4 + N * Co * 8 * D * H * W * 4
                      + wlhs.size * 2 + bias_bc.size * 4)
    cost = pl.CostEstimate(flops=flops, transcendentals=0,
                           bytes_accessed=bytes_accessed)

    out = pl.pallas_call(
        _conv_body,
        out_shape=jax.ShapeDtypeStruct((N, Co, 2 * D, 4 * H * W), jnp.float32),
        grid=(N, DT),
        in_specs=[
            pl.BlockSpec(memory_space=pl.ANY),
            pl.BlockSpec((4, Co, 12 * C), lambda n, t: (0, 0, 0)),
            pl.BlockSpec((Co, 512), lambda n, t: (0, 0)),
        ],
        out_specs=pl.BlockSpec((1, Co, 2 * TD, 4 * H * W),
                               lambda n, t: (n, 0, t, 0)),
        scratch_shapes=[
            pltpu.VMEM((TD + 2, C, H * W), jnp.float32),
            pltpu.VMEM((TD + 2, C, 608), jnp.bfloat16),
            pltpu.SemaphoreType.DMA((TD + 2,)),
        ],
        compiler_params=pltpu.CompilerParams(
            dimension_semantics=("arbitrary", "arbitrary"),
            vmem_limit_bytes=100 * 1024 * 1024),
        cost_estimate=cost,
    )(xv, wlhs, bias_bc)
    return out.reshape(N, Co, 2 * D, 2 * H, 2 * W)


# speedup vs baseline: 7.9798x; 7.9798x over previous
import jax
import jax.numpy as jnp
from jax.experimental import pallas as pl
from jax.experimental.pallas import tpu as pltpu


def _fused_body(x_hbm, w_ref, b_ref, o_ref, xbuf, sem):
    TD, _, H, _, W, Co2 = o_ref.shape
    C = xbuf.shape[-1]
    n = pl.program_id(0)
    t = pl.program_id(1)
    d0 = pl.multiple_of(t * TD, TD)

    cp = pltpu.make_async_copy(x_hbm.at[n, pl.ds(d0, TD + 2)], xbuf, sem)
    cp.start()
    cp.wait()

    x = xbuf[...]
    xw3 = jnp.concatenate(
        [x[:, :, 0:W, :], x[:, :, 1:W + 1, :], x[:, :, 2:W + 2, :]], axis=-1)

    bias = b_ref[...]
    for a in range(2):
        for b in range(2):
            patch = jnp.concatenate(
                [xw3[a:a + TD, b:b + H],
                 xw3[a:a + TD, b + 1:b + 1 + H],
                 xw3[a + 1:a + 1 + TD, b:b + H],
                 xw3[a + 1:a + 1 + TD, b + 1:b + 1 + H]],
                axis=-1).reshape(TD * H * W, 12 * C)
            y = jnp.dot(patch, w_ref[2 * a + b],
                        preferred_element_type=jnp.float32) + bias
            o_ref[:, a, :, b, :, :] = y.reshape(TD, H, W, Co2)


def _upsample_conv(x, we, bias2):
    N, D, H, W, C = x.shape
    Co2 = we.shape[2]
    TD = 4
    DT = D // TD
    xp = jnp.pad(x, ((0, 0), (1, 1), (1, 1), (1, 1), (0, 0)))

    flops = 2 * 4 * N * D * H * W * 12 * C * Co2
    bytes_accessed = (xp.size * 2 + N * D * H * W * 4 * Co2 * 4
                      + we.size * 2 + bias2.size * 4)
    cost = pl.CostEstimate(flops=flops, transcendentals=0,
                           bytes_accessed=bytes_accessed)

    out = pl.pallas_call(
        _fused_body,
        out_shape=jax.ShapeDtypeStruct((N * D, 2, H, 2, W, Co2), jnp.float32),
        grid=(N, DT),
        in_specs=[
            pl.BlockSpec(memory_space=pl.ANY),
            pl.BlockSpec((4, 12 * C, Co2), lambda n, t: (0, 0, 0)),
            pl.BlockSpec((1, Co2), lambda n, t: (0, 0)),
        ],
        out_specs=pl.BlockSpec((TD, 2, H, 2, W, Co2),
                               lambda n, t: (n * DT + t, 0, 0, 0, 0, 0)),
        scratch_shapes=[pltpu.VMEM((TD + 2, H + 2, W + 2, C), x.dtype),
                        pltpu.SemaphoreType.DMA],
        compiler_params=pltpu.CompilerParams(
            dimension_semantics=("arbitrary", "arbitrary"),
            vmem_limit_bytes=60 * 1024 * 1024),
        cost_estimate=cost,
    )(xp, we.astype(x.dtype), bias2)
    return out.reshape(N, 2 * D, 2 * H, 2 * W, Co2 // 2)


def kernel(hidden_states, we, bias2):
    x = jnp.transpose(hidden_states, (0, 2, 3, 4, 1)).astype(jnp.bfloat16)
    y = _upsample_conv(x, we, bias2)
    return jnp.transpose(y, (0, 4, 1, 2, 3))


# speedup vs baseline: 8.6918x; 1.0892x over previous
import jax
import jax.numpy as jnp
from jax.experimental import pallas as pl
from jax.experimental.pallas import tpu as pltpu


def _fused_body(x_hbm, w_ref, b_ref, o_ref, xbuf, sem):
    TD, _, H, _, W, Co2 = o_ref.shape
    C = xbuf.shape[-1]
    n = pl.program_id(0)
    t = pl.program_id(1)
    d0 = pl.multiple_of(t * TD, TD)

    cp = pltpu.make_async_copy(x_hbm.at[n, pl.ds(d0, TD + 2)], xbuf, sem)
    cp.start()
    cp.wait()

    x = xbuf[...]
    xw3 = jnp.concatenate(
        [x[:, :, 0:W, :], x[:, :, 1:W + 1, :], x[:, :, 2:W + 2, :]], axis=-1)

    bias = b_ref[...]
    for a in range(2):
        for b in range(2):
            patch = jnp.concatenate(
                [xw3[a:a + TD, b:b + H],
                 xw3[a:a + TD, b + 1:b + 1 + H],
                 xw3[a + 1:a + 1 + TD, b:b + H],
                 xw3[a + 1:a + 1 + TD, b + 1:b + 1 + H]],
                axis=-1).reshape(TD * H * W, 12 * C)
            y = jnp.dot(patch, w_ref[2 * a + b],
                        preferred_element_type=jnp.float32) + bias
            o_ref[:, a, :, b, :, :] = y.reshape(TD, H, W, Co2).astype(o_ref.dtype)


def _upsample_conv(x, we, bias2):
    N, D, H, W, C = x.shape
    Co2 = we.shape[2]
    TD = 4
    DT = D // TD
    xp = jnp.pad(x, ((0, 0), (1, 1), (1, 1), (1, 1), (0, 0)))

    flops = 2 * 4 * N * D * H * W * 12 * C * Co2
    bytes_accessed = (xp.size * 2 + N * D * H * W * 4 * Co2 * 4
                      + we.size * 2 + bias2.size * 4)
    cost = pl.CostEstimate(flops=flops, transcendentals=0,
                           bytes_accessed=bytes_accessed)

    out = pl.pallas_call(
        _fused_body,
        out_shape=jax.ShapeDtypeStruct((N * D, 2, H, 2, W, Co2), jnp.bfloat16),
        grid=(N, DT),
        in_specs=[
            pl.BlockSpec(memory_space=pl.ANY),
            pl.BlockSpec((4, 12 * C, Co2), lambda n, t: (0, 0, 0)),
            pl.BlockSpec((1, Co2), lambda n, t: (0, 0)),
        ],
        out_specs=pl.BlockSpec((TD, 2, H, 2, W, Co2),
                               lambda n, t: (n * DT + t, 0, 0, 0, 0, 0)),
        scratch_shapes=[pltpu.VMEM((TD + 2, H + 2, W + 2, C), x.dtype),
                        pltpu.SemaphoreType.DMA],
        compiler_params=pltpu.CompilerParams(
            dimension_semantics=("arbitrary", "arbitrary"),
            vmem_limit_bytes=60 * 1024 * 1024),
        cost_estimate=cost,
    )(xp, we.astype(x.dtype), bias2)
    return out.reshape(N, 2 * D, 2 * H, 2 * W, Co2 // 2)


def kernel(hidden_states, we, bias2):
    x = jnp.transpose(hidden_states, (0, 2, 3, 4, 1)).astype(jnp.bfloat16)
    y = _upsample_conv(x, we, bias2)
    return jnp.transpose(y, (0, 4, 1, 2, 3)).astype(jnp.float32)


# speedup vs baseline: 9.2988x; 1.0698x over previous
import jax
import jax.numpy as jnp
from jax.experimental import pallas as pl
from jax.experimental.pallas import tpu as pltpu


def _fused_body(x_hbm, w_ref, b_ref, o_ref, xbuf, sem):
    TD, _, H, _, W, Co2 = o_ref.shape
    C = xbuf.shape[-1]
    n = pl.program_id(0)
    t = pl.program_id(1)
    d0 = pl.multiple_of(t * TD, TD)

    cp = pltpu.make_async_copy(x_hbm.at[n, pl.ds(d0, TD + 2)], xbuf, sem)
    cp.start()
    cp.wait()

    x = xbuf[...]
    xw3 = jnp.concatenate(
        [x[:, :, 0:W, :], x[:, :, 1:W + 1, :], x[:, :, 2:W + 2, :]], axis=-1)

    bias = b_ref[...]
    for a in range(2):
        for b in range(2):
            patch = jnp.concatenate(
                [xw3[a:a + TD, b:b + H],
                 xw3[a:a + TD, b + 1:b + 1 + H],
                 xw3[a + 1:a + 1 + TD, b:b + H],
                 xw3[a + 1:a + 1 + TD, b + 1:b + 1 + H]],
                axis=-1).reshape(TD * H * W, 12 * C)
            y = jnp.dot(patch, w_ref[2 * a + b],
                        preferred_element_type=jnp.float32) + bias
            o_ref[:, a, :, b, :, :] = y.reshape(TD, H, W, Co2).astype(o_ref.dtype)


def _upsample_conv(x, we, bias2):
    N, D, H, W, C = x.shape
    Co2 = we.shape[2]
    TD = 8
    DT = D // TD
    xp = jnp.pad(x, ((0, 0), (1, 1), (1, 1), (1, 1), (0, 0)))

    flops = 2 * 4 * N * D * H * W * 12 * C * Co2
    bytes_accessed = (xp.size * 2 + N * D * H * W * 4 * Co2 * 4
                      + we.size * 2 + bias2.size * 4)
    cost = pl.CostEstimate(flops=flops, transcendentals=0,
                           bytes_accessed=bytes_accessed)

    out = pl.pallas_call(
        _fused_body,
        out_shape=jax.ShapeDtypeStruct((N * D, 2, H, 2, W, Co2), jnp.bfloat16),
        grid=(N, DT),
        in_specs=[
            pl.BlockSpec(memory_space=pl.ANY),
            pl.BlockSpec((4, 12 * C, Co2), lambda n, t: (0, 0, 0)),
            pl.BlockSpec((1, Co2), lambda n, t: (0, 0)),
        ],
        out_specs=pl.BlockSpec((TD, 2, H, 2, W, Co2),
                               lambda n, t: (n * DT + t, 0, 0, 0, 0, 0)),
        scratch_shapes=[pltpu.VMEM((TD + 2, H + 2, W + 2, C), x.dtype),
                        pltpu.SemaphoreType.DMA],
        compiler_params=pltpu.CompilerParams(
            dimension_semantics=("arbitrary", "arbitrary"),
            vmem_limit_bytes=60 * 1024 * 1024),
        cost_estimate=cost,
    )(xp, we.astype(x.dtype), bias2)
    return out.reshape(N, 2 * D, 2 * H, 2 * W, Co2 // 2)


def kernel(hidden_states, we, bias2):
    x = jnp.transpose(hidden_states, (0, 2, 3, 4, 1)).astype(jnp.bfloat16)
    y = _upsample_conv(x, we, bias2)
    return jnp.transpose(y, (0, 4, 1, 2, 3)).astype(jnp.float32)
